# baseline (speedup 1.0000x reference)
import functools

import jax
import jax.numpy as jnp
from jax import lax
from jax.experimental import pallas as pl
from jax.experimental.pallas import tpu as pltpu
from jax.experimental.pallas import tpu_sc as plsc

E = 320000
N = 10000
D = 128
NC = 2
NS = 16
NW = NC * NS
EPW = E // NW
ACCW = 30720
SLC = ACCW // NS

CH = 80
GPC = CH // 16
NCH = EPW // CH
RING = 6

_F32 = jnp.float32


def _dg(x, i):
    return x.at[i].get(mode="promise_in_bounds")


def _start_slot(f_hbm, vst_hbm, idx_hbm, wid, j, fb, vb, ib, sem):
    pltpu.async_copy(
        f_hbm.at[pl.ds((wid * EPW + j * CH) * D, CH * D)], fb, sem)
    pltpu.async_copy(
        vst_hbm.at[pl.ds((wid * EPW + j * CH) * 3, CH * 3)], vb, sem)
    pltpu.async_copy(
        idx_hbm.at[pl.ds(wid * EPW + j * CH, CH)], ib, sem)


def _wait_slot(f_hbm, vst_hbm, idx_hbm, wid, j, fb, vb, ib, sem):
    pltpu.make_async_copy(
        f_hbm.at[pl.ds((wid * EPW + j * CH) * D, CH * D)], fb, sem).wait()
    pltpu.make_async_copy(
        vst_hbm.at[pl.ds((wid * EPW + j * CH) * 3, CH * 3)], vb, sem).wait()
    pltpu.make_async_copy(
        idx_hbm.at[pl.ds(wid * EPW + j * CH, CH)], ib, sem).wait()


def _fused_body(f_hbm, vst_hbm, idx_hbm, w_hbm, b_hbm, out_hbm, refs):
    fbufs = refs[0:RING]
    vbufs = refs[RING:2 * RING]
    ibufs = refs[2 * RING:3 * RING]
    w_v, bb_v, acc_v, tmp_v, red_v = refs[3 * RING:3 * RING + 5]
    sems = refs[3 * RING + 5:3 * RING + 5 + RING]
    shared = refs[-1]

    c = lax.axis_index("c")
    s = lax.axis_index("s")
    wid = c * NS + s

    for b in range(RING):
        _start_slot(f_hbm, vst_hbm, idx_hbm, wid, b,
                    fbufs[b], vbufs[b], ibufs[b], sems[b])
    pltpu.sync_copy(w_hbm, w_v)
    pltpu.sync_copy(b_hbm, bb_v)

    zeros = jnp.zeros((16,), _F32)

    def _zero(i, _):
        acc_v[pl.ds(i * 16, 16)] = zeros
        return ()

    lax.fori_loop(0, ACCW // 16, _zero, (), unroll=4)

    b_val = bb_v[...][0]
    wv = [w_v[pl.ds(j * 16, 16)] for j in range(8)]
    iota = lax.iota(jnp.int32, 16)
    b_vec = jnp.zeros((16,), _F32) + b_val
    iota3 = iota * 3
    is15 = iota == 15
    fifteen = jnp.full((16,), 15, jnp.int32)
    shifts = tuple((jnp.maximum(iota - d, 0), iota >= d) for d in (1, 2, 4, 8))

    def _make_chunk(fbuf, vbuf, ibuf):
        def _group(gg, _):
            ids = ibuf[pl.ds(gg * 16, 16)]
            end = (ids != _dg(ids, jnp.minimum(iota + 1, 15))) | is15
            masks = tuple(((ids == _dg(ids, sh)) & valid, sh)
                          for sh, valid in shifts)
            pos0 = ids * 3

            kbase = gg * (16 * D)
            dot = b_vec
            for e in range(16):
                off = kbase + e * D
                t = [fbuf[pl.ds(off + 16 * j, 16)] * wv[j] for j in range(8)]
                t4 = [t[2 * i] + t[2 * i + 1] for i in range(4)]
                r = (t4[0] + t4[1]) + (t4[2] + t4[3])
                tot = _dg(plsc.cumsum(r), fifteen)
                dot = jnp.where(iota == e, tot, dot)

            def _chan(ch):
                sv = dot * plsc.load_gather(vbuf, [iota3 + (gg * 48 + ch)])
                for m, sh in masks:
                    sv = sv + jnp.where(m, _dg(sv, sh), 0.0)
                plsc.addupdate_scatter(acc_v, [pos0 + ch], sv, mask=end)

            _chan(0)
            _chan(1)
            _chan(2)
            return ()

        return _group

    def _step(gr, _):
        for b in range(RING):
            j = RING * gr + b
            _wait_slot(f_hbm, vst_hbm, idx_hbm, wid, j,
                       fbufs[b], vbufs[b], ibufs[b], sems[b])
            lax.fori_loop(0, GPC, _make_chunk(fbufs[b], vbufs[b], ibufs[b]), ())

            @pl.when(j + RING < NCH)
            def _():
                _start_slot(f_hbm, vst_hbm, idx_hbm, wid, j + RING,
                            fbufs[b], vbufs[b], ibufs[b], sems[b])
        return ()

    lax.fori_loop(0, NCH // RING, _step, ())
    for b in range(NCH % RING):
        j = (NCH // RING) * RING + b
        _wait_slot(f_hbm, vst_hbm, idx_hbm, wid, j,
                   fbufs[b], vbufs[b], ibufs[b], sems[b])
        lax.fori_loop(0, GPC, _make_chunk(fbufs[b], vbufs[b], ibufs[b]), ())

    pltpu.sync_copy(acc_v, shared.at[s])
    plsc.subcore_barrier()

    def _rzero(i, _):
        red_v[pl.ds(i * 16, 16)] = zeros
        return ()

    lax.fori_loop(0, SLC // 16, _rzero, (), unroll=4)

    def _red(p, _):
        pltpu.sync_copy(shared.at[p, pl.ds(s * SLC, SLC)], tmp_v)

        def _add(i, _):
            red_v[pl.ds(i * 16, 16)] += tmp_v[pl.ds(i * 16, 16)]
            return ()

        lax.fori_loop(0, SLC // 16, _add, (), unroll=4)
        return ()

    lax.fori_loop(0, NS, _red, ())
    pltpu.sync_copy(red_v, out_hbm.at[c, pl.ds(s * SLC, SLC)])


_SCRATCH = (
    [pltpu.VMEM((CH * D,), _F32) for _ in range(RING)]
    + [pltpu.VMEM((CH * 3,), _F32) for _ in range(RING)]
    + [pltpu.VMEM((CH,), jnp.int32) for _ in range(RING)]
    + [
        pltpu.VMEM((D,), _F32),
        pltpu.VMEM((16,), _F32),
        pltpu.VMEM((ACCW,), _F32),
        pltpu.VMEM((SLC,), _F32),
        pltpu.VMEM((SLC,), _F32),
    ]
    + [pltpu.SemaphoreType.DMA for _ in range(RING)]
    + [pltpu.VMEM_SHARED((NS, ACCW), _F32)]
)


@functools.partial(
    pl.kernel,
    out_type=jax.ShapeDtypeStruct((NC, ACCW), _F32),
    mesh=plsc.VectorSubcoreMesh(core_axis_name="c", subcore_axis_name="s"),
    compiler_params=pltpu.CompilerParams(needs_layout_passes=False),
    scratch_types=_SCRATCH,
)
def _sc_fused(f_hbm, vst_hbm, idx_hbm, w_hbm, b_hbm, out_hbm, *refs):
    _fused_body(f_hbm, vst_hbm, idx_hbm, w_hbm, b_hbm, out_hbm, refs)


def _combine_body(p_ref, o_ref):
    o_ref[...] = jnp.sum(p_ref[...], axis=0, keepdims=True)


def _tc_combine(partial):
    return pl.pallas_call(
        _combine_body,
        out_shape=jax.ShapeDtypeStruct((1, ACCW), _F32),
    )(partial)


def kernel(forces, V_st, idx_t, W, b):
    partial = _sc_fused(forces.reshape(-1), V_st.reshape(-1),
                        idx_t.astype(jnp.int32), W.reshape(-1),
                        jnp.concatenate([b, jnp.zeros((15,), jnp.float32)]))
    out = _tc_combine(partial)
    return out[0, : N * 3].reshape(N, 3)
